# baseline (speedup 1.0000x reference)
import functools

import jax
import jax.numpy as jnp
from jax import lax
from jax.experimental import pallas as pl
from jax.experimental.pallas import tpu as pltpu
from jax.experimental.pallas import tpu_sc as plsc

_V = 1_000_000
_D = 32
_B = 16384
_NC = 2
_NS = 16
_NW = _NC * _NS
_BPW = _B // _NW
_NLB = (_V + 127) // 128
_MPW = (_NLB + _NW - 1) // _NW
_HCAP = 768
_SROWS = _B + 128
_NBUF = 8


def _dedup_body(tt_hbm, idx_hbm, s_hbm, idx_v, hv, hb, counts_s, counts_v,
                hitlist, slots_l, slots_b,
                k0, k1b, k2b, k3, k4, k5, k6, k7,
                pendbuf, pendidx, isem, fsem, wsem):
    bufs = [k0, k1b, k2b, k3, k4, k5, k6, k7]
    wid = lax.axis_index("s") * _NC + lax.axis_index("c")
    iota = lax.iota(jnp.int32, 16)
    zeros16 = jnp.zeros((16,), jnp.int32)
    lane0 = iota < 1

    pltpu.async_copy(idx_hbm.at[pl.ds(0, _B)], idx_v, isem).wait()

    for c in range(16):
        counts_v[pl.ds(c * 16, 16)] = zeros16
        hitlist[pl.ds(c * 16, 16)] = zeros16

    def zero_counts(i):
        counts_s[i] = jnp.int32(0)

    pl.loop(0, _MPW)(zero_counts)

    def scan(g, cnt):
        v16 = idx_v[pl.ds(g * 16, 16)]
        mine = ((v16 >> 7) & 31) == wid
        inc = jnp.where(mine, 1, 0).astype(jnp.int32)
        pos = cnt + plsc.cumsum(inc) - 1
        plsc.store_scatter(hv, [pos], v16, mask=mine)
        plsc.store_scatter(hb, [pos], iota + g * 16, mask=mine)
        return cnt + plsc.all_reduce_population_count(mine)[0]

    cnt = pl.loop(0, _B // 16, init_carry=jnp.int32(0))(scan)
    nhc = (cnt + 15) >> 4

    def hist(c):
        valid = (c * 16 + iota) < cnt
        k16 = hv[pl.ds(c * 16, 16)] >> 12
        plsc.addupdate_scatter(
            counts_v, [k16], jnp.where(valid, 1, 0).astype(jnp.int32),
            mask=valid)

    pl.loop(0, nhc)(hist)

    def blocks(c, acc):
        n16 = counts_v[pl.ds(c * 16, 16)]
        has = n16 > 0
        pos = acc + plsc.cumsum(jnp.where(has, 1, 0).astype(jnp.int32)) - 1
        plsc.store_scatter(hitlist, [pos], iota + c * 16, mask=has)
        return acc + plsc.all_reduce_population_count(has)[0]

    nhit = pl.loop(0, 16, init_carry=jnp.int32(0))(blocks)
    nchunk = (nhit + 15) >> 4

    def place(c):
        hv16 = hv[pl.ds(c * 16, 16)]
        hb16 = hb[pl.ds(c * 16, 16)]
        for j in range(16):
            @pl.when(c * 16 + j < cnt)
            def _(v=hv16[j], b=hb16[j]):
                k = v >> 12
                sl = counts_s[k]
                counts_s[k] = sl + 1
                ok = lane0 & lax.broadcast(sl < 16, (16,))
                at = lax.broadcast(k * 16 + sl, (16,))
                plsc.store_scatter(slots_l, [at],
                                   lax.broadcast(v & 127, (16,)), mask=ok)
                plsc.store_scatter(slots_b, [at],
                                   lax.broadcast(b, (16,)), mask=ok)

    pl.loop(0, nhc)(place)

    def fetch(buf, m):
        lb = pl.multiple_of(((m << 5) + wid) << 7, 128)
        pltpu.async_copy(tt_hbm.at[:, pl.ds(lb, 128)], buf, fsem)

    def drain_f(buf):
        pltpu.make_async_copy(tt_hbm.at[:, pl.ds(0, 128)], buf, fsem).wait()

    def flush_maybe(pend, flushed, fi, fd):
        do = (pend - flushed) >= 16

        @pl.when(do & ((fi - fd) >= 2))
        def _():
            pltpu.make_async_copy(
                pendbuf.at[pl.ds(0, 16), :],
                s_hbm.at[pl.ds(_B, 16), :], wsem).wait()

        @pl.when(do)
        def _():
            off = pl.multiple_of(flushed & 31, 16)
            idxvec = pendidx[pl.ds(off, 16)]
            pltpu.async_copy(
                pendbuf.at[pl.ds(off, 16), :],
                s_hbm.at[plsc.Indices(idxvec, ignored_value=-1)],
                wsem)

        fd = jnp.where(do & ((fi - fd) >= 2), fd + 1, fd)
        flushed = jnp.where(do, flushed + 16, flushed)
        fi = jnp.where(do, fi + 1, fi)
        return flushed, fi, fd

    def process(buf, m, pend):
        n = counts_s[m]
        l16 = slots_l[pl.ds(m * 16, 16)]
        b16 = slots_b[pl.ds(m * 16, 16)]
        lmask = iota < lax.broadcast(n, (16,))
        ring = (lax.broadcast(pend, (16,)) + iota) & 31
        for d in range(_D):
            dd = lax.broadcast(jnp.int32(d), (16,))
            vals = plsc.load_gather(buf, [dd, l16], mask=lmask)
            plsc.store_scatter(pendbuf, [ring, dd], vals, mask=lmask)
        plsc.store_scatter(pendidx, [ring], b16, mask=lmask)
        return pend + n

    hl0 = hitlist[pl.ds(0, 16)]
    for j in range(_NBUF):
        fetch(bufs[j], hl0[j])

    def chunk(c, carry):
        pend, flushed, fi, fd = carry
        hl16 = hitlist[pl.ds(c * 16, 16)]
        cn = jnp.minimum(c + 1, 15)
        hlnext = hitlist[pl.ds(cn * 16, 16)]
        for j in range(16):
            buf = bufs[j % _NBUF]
            drain_f(buf)
            pend = process(buf, hl16[j], pend)
            flushed, fi, fd = flush_maybe(pend, flushed, fi, fd)
            m2 = hl16[j + _NBUF] if j < _NBUF else hlnext[j - _NBUF]
            fetch(buf, m2)
        return pend, flushed, fi, fd

    pend, flushed, fi, fd = pl.loop(
        0, nchunk,
        init_carry=(jnp.int32(0), jnp.int32(0), jnp.int32(0), jnp.int32(0)),
    )(chunk)

    r = (16 - (pend & 15)) & 15
    ringp = (lax.broadcast(pend, (16,)) + iota) & 31
    plsc.store_scatter(pendidx, [ringp], lax.broadcast(_B, (16,)) + iota,
                       mask=iota < lax.broadcast(r, (16,)))
    pend = pend + r
    for _ in range(2):
        flushed, fi, fd = flush_maybe(pend, flushed, fi, fd)
    for _ in range(4):
        @pl.when(fi - fd > 0)
        def _():
            pltpu.make_async_copy(
                pendbuf.at[pl.ds(0, 16), :],
                s_hbm.at[pl.ds(_B, 16), :], wsem).wait()

        fd = jnp.where(fi - fd > 0, fd + 1, fd)
    for j in range(_NBUF):
        drain_f(bufs[j])


def _permute_body(s_hbm, out_hbm, sv, slab, sem):
    wid = lax.axis_index("s") * _NC + lax.axis_index("c")
    base = wid * _BPW
    iota = lax.iota(jnp.int32, 16)
    pltpu.async_copy(s_hbm.at[pl.ds(base, _BPW), :], sv, sem).wait()
    for d in range(_D):
        dd = lax.broadcast(jnp.int32(d), (16,))
        for c2 in range(_BPW // 16):
            rows = iota + c2 * 16
            vals = plsc.load_gather(sv, [rows, dd])
            slab[d, pl.ds(c2 * 16, 16)] = vals
    pltpu.sync_copy(slab, out_hbm.at[:, pl.ds(base, _BPW)])


@jax.jit
def kernel(indices, table):
    mesh = plsc.VectorSubcoreMesh(
        core_axis_name="c", subcore_axis_name="s",
        num_cores=_NC, num_subcores=_NS,
    )
    run1 = pl.kernel(
        _dedup_body,
        out_type=jax.ShapeDtypeStruct((_SROWS, 128), jnp.float32),
        mesh=mesh,
        scratch_types=(
            [
                pltpu.VMEM((_B,), jnp.int32),
                pltpu.VMEM((_HCAP,), jnp.int32),
                pltpu.VMEM((_HCAP,), jnp.int32),
                pltpu.SMEM((256,), jnp.int32),
                pltpu.VMEM((256,), jnp.int32),
                pltpu.VMEM((256,), jnp.int32),
                pltpu.VMEM((_MPW * 16 + 16,), jnp.int32),
                pltpu.VMEM((_MPW * 16 + 16,), jnp.int32),
            ]
            + [pltpu.VMEM((_D, 128), jnp.float32) for _ in range(_NBUF)]
            + [
                pltpu.VMEM((32, 128), jnp.float32),
                pltpu.VMEM((32,), jnp.int32),
                pltpu.SemaphoreType.DMA,
                pltpu.SemaphoreType.DMA,
                pltpu.SemaphoreType.DMA,
            ]
        ),
        compiler_params=pltpu.CompilerParams(needs_layout_passes=False),
    )
    run2 = pl.kernel(
        _permute_body,
        out_type=jax.ShapeDtypeStruct((_D, _B), jnp.float32),
        mesh=mesh,
        scratch_types=[
            pltpu.VMEM((_BPW, 128), jnp.float32),
            pltpu.VMEM((_D, _BPW), jnp.float32),
            pltpu.SemaphoreType.DMA,
        ],
        compiler_params=pltpu.CompilerParams(needs_layout_passes=False),
    )
    s = run1(table.T, indices)
    out_t = run2(s)
    return out_t.T


# speedup vs baseline: 1.0314x; 1.0314x over previous
import functools

import jax
import jax.numpy as jnp
from jax import lax
from jax.experimental import pallas as pl
from jax.experimental.pallas import tpu as pltpu
from jax.experimental.pallas import tpu_sc as plsc

_V = 1_000_000
_D = 32
_B = 16384
_NC = 2
_NS = 16
_NW = _NC * _NS
_BPW = _B // _NW
_NLB = (_V + 127) // 128
_MPW = (_NLB + _NW - 1) // _NW
_HCAP = 768
_SROWS = _B + 128
_NBUF = 8


def _dedup_body(tt_hbm, idx_hbm, s_hbm, idx_v, hv, hb, counts_s, counts_v,
                hitlist, slots_l, slots_b,
                k0, k1b, k2b, k3, k4, k5, k6, k7,
                pendbuf, pendidx, isem, fsem, wsem):
    bufs = [k0, k1b, k2b, k3, k4, k5, k6, k7]
    wid = lax.axis_index("s") * _NC + lax.axis_index("c")
    iota = lax.iota(jnp.int32, 16)
    zeros16 = jnp.zeros((16,), jnp.int32)
    lane0 = iota < 1

    pltpu.async_copy(idx_hbm.at[pl.ds(0, _B)], idx_v, isem).wait()

    for c in range(16):
        counts_v[pl.ds(c * 16, 16)] = zeros16
        hitlist[pl.ds(c * 16, 16)] = zeros16

    def zero_counts(i):
        counts_s[i] = jnp.int32(0)

    pl.loop(0, _MPW)(zero_counts)

    def scan(g, cnt):
        v16 = idx_v[pl.ds(g * 16, 16)]
        mine = ((v16 >> 7) & 31) == wid
        inc = jnp.where(mine, 1, 0).astype(jnp.int32)
        pos = cnt + plsc.cumsum(inc) - 1
        plsc.store_scatter(hv, [pos], v16, mask=mine)
        plsc.store_scatter(hb, [pos], iota + g * 16, mask=mine)
        return cnt + plsc.all_reduce_population_count(mine)[0]

    cnt = pl.loop(0, _B // 16, init_carry=jnp.int32(0))(scan)
    nhc = (cnt + 15) >> 4

    def hist(c):
        valid = (c * 16 + iota) < cnt
        k16 = hv[pl.ds(c * 16, 16)] >> 12
        plsc.addupdate_scatter(
            counts_v, [k16], jnp.where(valid, 1, 0).astype(jnp.int32),
            mask=valid)

    pl.loop(0, nhc)(hist)

    def blocks(c, acc):
        n16 = counts_v[pl.ds(c * 16, 16)]
        has = n16 > 0
        pos = acc + plsc.cumsum(jnp.where(has, 1, 0).astype(jnp.int32)) - 1
        plsc.store_scatter(hitlist, [pos], iota + c * 16, mask=has)
        return acc + plsc.all_reduce_population_count(has)[0]

    nhit = pl.loop(0, 16, init_carry=jnp.int32(0))(blocks)
    nchunk = (nhit + 15) >> 4

    def place(c):
        hv16 = hv[pl.ds(c * 16, 16)]
        hb16 = hb[pl.ds(c * 16, 16)]
        for j in range(16):
            @pl.when(c * 16 + j < cnt)
            def _(v=hv16[j], b=hb16[j]):
                k = v >> 12
                sl = counts_s[k]
                counts_s[k] = sl + 1
                ok = lane0 & lax.broadcast(sl < 16, (16,))
                at = lax.broadcast(k * 16 + sl, (16,))
                plsc.store_scatter(slots_l, [at],
                                   lax.broadcast(v & 127, (16,)), mask=ok)
                plsc.store_scatter(slots_b, [at],
                                   lax.broadcast(b, (16,)), mask=ok)

    pl.loop(0, nhc)(place)

    def fetch(buf, m):
        lb = pl.multiple_of(((m << 5) + wid) << 7, 128)
        pltpu.async_copy(tt_hbm.at[:, pl.ds(lb, 128)], buf, fsem)

    def drain_f(buf):
        pltpu.make_async_copy(tt_hbm.at[:, pl.ds(0, 128)], buf, fsem).wait()

    def flush_maybe(pend, flushed, fi, fd):
        do = (pend - flushed) >= 16

        @pl.when(do & ((fi - fd) >= 2))
        def _():
            pltpu.make_async_copy(
                pendbuf.at[pl.ds(0, 16), :],
                s_hbm.at[pl.ds(_B, 16), :], wsem).wait()

        @pl.when(do)
        def _():
            off = pl.multiple_of(flushed & 31, 16)
            idxvec = pendidx[pl.ds(off, 16)]
            pltpu.async_copy(
                pendbuf.at[pl.ds(off, 16), :],
                s_hbm.at[plsc.Indices(idxvec, ignored_value=-1)],
                wsem)

        fd = jnp.where(do & ((fi - fd) >= 2), fd + 1, fd)
        flushed = jnp.where(do, flushed + 16, flushed)
        fi = jnp.where(do, fi + 1, fi)
        return flushed, fi, fd

    def process(buf, m, pend):
        n = counts_s[m]
        l16 = slots_l[pl.ds(m * 16, 16)]
        b16 = slots_b[pl.ds(m * 16, 16)]
        lmask = iota < lax.broadcast(n, (16,))
        ring = (lax.broadcast(pend, (16,)) + iota) & 31
        for d in range(_D):
            dd = lax.broadcast(jnp.int32(d), (16,))
            vals = plsc.load_gather(buf, [dd, l16], mask=lmask)
            plsc.store_scatter(pendbuf, [ring, dd], vals, mask=lmask)
        plsc.store_scatter(pendidx, [ring], b16, mask=lmask)
        return pend + n

    hl0 = hitlist[pl.ds(0, 16)]
    for j in range(_NBUF):
        fetch(bufs[j], hl0[j])

    def chunk(c, carry):
        pend, flushed, fi, fd = carry
        hl16 = hitlist[pl.ds(c * 16, 16)]
        cn = jnp.minimum(c + 1, 15)
        hlnext = hitlist[pl.ds(cn * 16, 16)]
        for j in range(16):
            buf = bufs[j % _NBUF]
            drain_f(buf)
            pend = process(buf, hl16[j], pend)
            flushed, fi, fd = flush_maybe(pend, flushed, fi, fd)
            m2 = hl16[j + _NBUF] if j < _NBUF else hlnext[j - _NBUF]
            fetch(buf, m2)
        return pend, flushed, fi, fd

    pend, flushed, fi, fd = pl.loop(
        0, nchunk,
        init_carry=(jnp.int32(0), jnp.int32(0), jnp.int32(0), jnp.int32(0)),
    )(chunk)

    r = (16 - (pend & 15)) & 15
    ringp = (lax.broadcast(pend, (16,)) + iota) & 31
    plsc.store_scatter(pendidx, [ringp], lax.broadcast(_B, (16,)) + iota,
                       mask=iota < lax.broadcast(r, (16,)))
    pend = pend + r
    for _ in range(2):
        flushed, fi, fd = flush_maybe(pend, flushed, fi, fd)
    for _ in range(4):
        @pl.when(fi - fd > 0)
        def _():
            pltpu.make_async_copy(
                pendbuf.at[pl.ds(0, 16), :],
                s_hbm.at[pl.ds(_B, 16), :], wsem).wait()

        fd = jnp.where(fi - fd > 0, fd + 1, fd)
    for j in range(_NBUF):
        drain_f(bufs[j])


def _permute_body(s_hbm, out_hbm, sv, slab, sem):
    wid = lax.axis_index("s") * _NC + lax.axis_index("c")
    base = wid * _BPW
    iota = lax.iota(jnp.int32, 16)
    pltpu.async_copy(s_hbm.at[pl.ds(base, _BPW), :], sv, sem).wait()
    for d in range(_D):
        dd = lax.broadcast(jnp.int32(d), (16,))

        def col(c2, d=d, dd=dd):
            rows = iota + c2 * 16
            vals = plsc.load_gather(sv, [rows, dd])
            slab[d, pl.ds(c2 * 16, 16)] = vals

        pl.loop(0, _BPW // 16)(col)
    pltpu.sync_copy(slab, out_hbm.at[:, pl.ds(base, _BPW)])


@jax.jit
def kernel(indices, table):
    mesh = plsc.VectorSubcoreMesh(
        core_axis_name="c", subcore_axis_name="s",
        num_cores=_NC, num_subcores=_NS,
    )
    run1 = pl.kernel(
        _dedup_body,
        out_type=jax.ShapeDtypeStruct((_SROWS, 128), jnp.float32),
        mesh=mesh,
        scratch_types=(
            [
                pltpu.VMEM((_B,), jnp.int32),
                pltpu.VMEM((_HCAP,), jnp.int32),
                pltpu.VMEM((_HCAP,), jnp.int32),
                pltpu.SMEM((256,), jnp.int32),
                pltpu.VMEM((256,), jnp.int32),
                pltpu.VMEM((256,), jnp.int32),
                pltpu.VMEM((_MPW * 16 + 16,), jnp.int32),
                pltpu.VMEM((_MPW * 16 + 16,), jnp.int32),
            ]
            + [pltpu.VMEM((_D, 128), jnp.float32) for _ in range(_NBUF)]
            + [
                pltpu.VMEM((32, 128), jnp.float32),
                pltpu.VMEM((32,), jnp.int32),
                pltpu.SemaphoreType.DMA,
                pltpu.SemaphoreType.DMA,
                pltpu.SemaphoreType.DMA,
            ]
        ),
        compiler_params=pltpu.CompilerParams(needs_layout_passes=False),
    )
    run2 = pl.kernel(
        _permute_body,
        out_type=jax.ShapeDtypeStruct((_D, _B), jnp.float32),
        mesh=mesh,
        scratch_types=[
            pltpu.VMEM((_BPW, 128), jnp.float32),
            pltpu.VMEM((_D, _BPW), jnp.float32),
            pltpu.SemaphoreType.DMA,
        ],
        compiler_params=pltpu.CompilerParams(needs_layout_passes=False),
    )
    s = run1(table.T, indices)
    out_t = run2(s)
    return out_t.T


# speedup vs baseline: 1.1457x; 1.1108x over previous
import functools

import jax
import jax.numpy as jnp
from jax import lax
from jax.experimental import pallas as pl
from jax.experimental.pallas import tpu as pltpu
from jax.experimental.pallas import tpu_sc as plsc

_D = 32
_B = 16384
_NC = 2
_NS = 16
_NW = _NC * _NS
_BPW = _B // _NW
_NBUF = 16
_NG = _BPW // 16


def _gather_body(tt_hbm, idx_hbm, out_hbm, idx_v,
                 b0, b1, b2, b3, b4, b5, b6, b7,
                 b8, b9, b10, b11, b12, b13, b14, b15, slab_v, isem, gsem):
    bufs = [b0, b1, b2, b3, b4, b5, b6, b7,
            b8, b9, b10, b11, b12, b13, b14, b15]
    wid = lax.axis_index("s") * _NC + lax.axis_index("c")
    base = wid * _BPW
    pltpu.async_copy(idx_hbm.at[pl.ds(base, _BPW)], idx_v, isem).wait()

    row_lo = lax.iota(jnp.int32, 16)
    row_hi = row_lo + 16

    def fetch(buf, v):
        lb = pl.multiple_of((v >> 7) << 7, 128)
        for sb in range(4):
            pltpu.async_copy(
                tt_hbm.at[pl.ds(8 * sb, 8), pl.ds(lb, 128)],
                buf.at[pl.ds(8 * sb, 8), :],
                gsem,
            )

    def drain(buf):
        for sb in range(4):
            pltpu.make_async_copy(
                tt_hbm.at[pl.ds(8 * sb, 8), pl.ds(0, 128)],
                buf.at[pl.ds(8 * sb, 8), :],
                gsem,
            ).wait()

    def extract(buf, v, col):
        l = lax.broadcast(v & 127, (16,))
        lo = plsc.load_gather(buf, [row_lo, l])
        hi = plsc.load_gather(buf, [row_hi, l])
        c16 = lax.broadcast(col, (16,))
        plsc.store_scatter(slab_v, [row_lo, c16], lo)
        plsc.store_scatter(slab_v, [row_hi, c16], hi)

    vs0 = idx_v[pl.ds(0, 16)]
    for j in range(16):
        fetch(bufs[j], vs0[j])

    def group(g):
        off = g * 16
        vs = idx_v[pl.ds(off, 16)]
        gn = jnp.minimum(g + 1, _NG - 1)
        vs1 = idx_v[pl.ds(gn * 16, 16)]
        for j in range(16):
            buf = bufs[j]
            drain(buf)
            extract(buf, vs[j], off + j)

            @pl.when(g + 1 < _NG)
            def _(buf=buf, vnext=vs1[j]):
                fetch(buf, vnext)

    pl.loop(0, _NG)(group)
    pltpu.sync_copy(slab_v, out_hbm.at[:, pl.ds(base, _BPW)])


@jax.jit
def kernel(indices, table):
    mesh = plsc.VectorSubcoreMesh(
        core_axis_name="c", subcore_axis_name="s",
        num_cores=_NC, num_subcores=_NS,
    )
    run = pl.kernel(
        _gather_body,
        out_type=jax.ShapeDtypeStruct((_D, _B), jnp.float32),
        mesh=mesh,
        scratch_types=(
            [pltpu.VMEM((_BPW,), jnp.int32)]
            + [pltpu.VMEM((_D, 128), jnp.float32) for _ in range(_NBUF)]
            + [
                pltpu.VMEM((_D, _BPW), jnp.float32),
                pltpu.SemaphoreType.DMA,
                pltpu.SemaphoreType.DMA,
            ]
        ),
        compiler_params=pltpu.CompilerParams(needs_layout_passes=False),
    )
    out_t = run(table.T, indices)
    return out_t.T
